# baseline (speedup 1.0000x reference)
import functools

import jax
import jax.numpy as jnp
import numpy as np
from jax import lax
from jax.experimental import pallas as pl
from jax.experimental.pallas import tpu as pltpu
from jax.experimental.pallas import tpu_sc as plsc

B = 16384
F = 26
V = 100000
D = 16
ROWS = F * (V + 1)

NC = 2
NS = 16
NW = NC * NS
NIDX = B * F
CHUNK = 128
PER_W = NIDX // NW
NCH_W = PER_W // CHUNK
NCH_TOTAL = NIDX // CHUNK


GROUP = 13
GROWS = GROUP * CHUNK
NGRP = NCH_W // GROUP

VW = 2048
NFULL = (V + 1) // VW
VTAIL = (V + 1) - NFULL * VW
CPF = NFULL + 1
FSTRIDE = CPF * VW
ROWSP = F * FSTRIDE
NITEMS = F * CPF
K1_ITERS = -(-NITEMS // NW)


def _transpose_body(nat_hbm, tail_hbm, lin_out, ibuf, obuf, osem0, osem1):
    wid = lax.axis_index("s") * NC + lax.axis_index("c")
    row16 = lax.iota(jnp.int32, 16)
    osems = (osem0, osem1)

    def dst_off(item):
        return (item // CPF * FSTRIDE + item % CPF * VW) * D

    def load_input(item):
        f = item // CPF
        c = item % CPF

        @pl.when(c < NFULL)
        def _():
            pltpu.sync_copy(nat_hbm.at[pl.ds(f * D, D), pl.ds(c * VW, VW)],
                            ibuf)

        @pl.when(c == NFULL)
        def _():
            pltpu.sync_copy(tail_hbm.at[pl.ds(f * D, D)], ibuf)

    def transpose_into(p):
        def xpose(v, cv):
            row = plsc.load_gather(ibuf, [row16, cv])
            obuf[p, pl.ds(v * 16, 16)] = row
            return cv + 1

        plsc.parallel_loop(0, VW, unroll=8,
                           carry=jnp.zeros((16,), jnp.int32))(xpose)

    def step(i, p):
        item = wid + i * NW
        prev = item - 2 * NW

        @pl.when((prev >= 0) & (prev < NITEMS))
        def _():
            pltpu.make_async_copy(obuf.at[p],
                                  lin_out.at[pl.ds(dst_off(prev), VW * D)],
                                  osems[p]).wait()

        @pl.when(item < NITEMS)
        def _():
            load_input(item)
            transpose_into(p)
            pltpu.async_copy(obuf.at[p],
                             lin_out.at[pl.ds(dst_off(item), VW * D)],
                             osems[p])

    def outer(io, carry):
        for h in range(2):
            step(2 * io + h, h)
        return carry

    lax.fori_loop(0, (K1_ITERS + 2) // 2, outer, 0)


@jax.jit
def _sc_transpose(nat, tail):
    mesh = plsc.VectorSubcoreMesh(core_axis_name="c", subcore_axis_name="s")
    return pl.kernel(
        _transpose_body,
        out_type=jax.ShapeDtypeStruct((ROWSP * D,), jnp.float32),
        mesh=mesh,
        compiler_params=pltpu.CompilerParams(needs_layout_passes=False),
        scratch_types=[
            pltpu.VMEM((D, VW), jnp.float32),
            pltpu.VMEM((2, VW * D), jnp.float32),
            pltpu.SemaphoreType.DMA,
            pltpu.SemaphoreType.DMA,
        ],
    )(nat, tail)


def _gather_body(gidx_hbm, emb_hbm, fo_hbm, emb_out, fo_out,
                 idx_v, ebig, fbig, gsem0, gsem1, wsem0, wsem1):
    wid = lax.axis_index("s") * NC + lax.axis_index("c")
    base_row = wid * PER_W
    pltpu.sync_copy(gidx_hbm.at[pl.ds(wid * NGRP, NGRP)], idx_v)

    gsems = (gsem0, gsem1)
    wsems = (wsem0, wsem1)

    def issue_gathers(g, p):
        pltpu.async_copy(emb_hbm.at[idx_v.at[g]], ebig.at[p], gsems[p])
        pltpu.async_copy(fo_hbm.at[idx_v.at[g]], fbig.at[p], gsems[p])

    def wait_gathers(g, p):
        pltpu.make_async_copy(emb_hbm.at[idx_v.at[g]], ebig.at[p],
                              gsems[p]).wait()
        pltpu.make_async_copy(fo_hbm.at[idx_v.at[g]], fbig.at[p],
                              gsems[p]).wait()

    def issue_wb(g, p):
        start = base_row + g * GROWS
        pltpu.async_copy(ebig.at[p], emb_out.at[pl.ds(start, GROWS)], wsems[p])
        pltpu.async_copy(fbig.at[p], fo_out.at[pl.ds(start, GROWS)], wsems[p])

    def wait_wb(g, p):
        start = base_row + g * GROWS
        pltpu.make_async_copy(ebig.at[p], emb_out.at[pl.ds(start, GROWS)],
                              wsems[p]).wait()
        pltpu.make_async_copy(fbig.at[p], fo_out.at[pl.ds(start, GROWS)],
                              wsems[p]).wait()

    for g in range(2):
        issue_gathers(g, g)
        wait_gathers(g, g)
        issue_wb(g, g)

    def outer(go, carry):
        for p in range(2):
            g = 2 * go + 2 + p
            wait_wb(g - 2, p)
            issue_gathers(g, p)
            wait_gathers(g, p)
            issue_wb(g, p)
        return carry

    lax.fori_loop(0, (NGRP - 2) // 2, outer, 0)

    for p in range(2):
        wait_wb(NGRP - 2 + p, p)


@jax.jit
def _sc_gather(gidx, emb_flat, fo_flat):
    mesh = plsc.VectorSubcoreMesh(core_axis_name="c", subcore_axis_name="s")
    return pl.kernel(
        _gather_body,
        out_type=(
            jax.ShapeDtypeStruct((NIDX, D), jnp.float32),
            jax.ShapeDtypeStruct((NIDX,), jnp.float32),
        ),
        mesh=mesh,
        compiler_params=pltpu.CompilerParams(use_tc_tiling_on_sc=False),
        scratch_types=[
            pltpu.VMEM((NGRP, GROWS), jnp.int32),
            pltpu.VMEM((2, GROWS, D), jnp.float32),
            pltpu.VMEM((2, GROWS), jnp.float32),
            pltpu.SemaphoreType.DMA,
            pltpu.SemaphoreType.DMA,
            pltpu.SemaphoreType.DMA,
            pltpu.SemaphoreType.DMA,
        ],
    )(gidx, emb_flat, fo_flat)


BB = 2048


def _mlp_body(emb_ref, fo_ref, s_ref, w1_ref, b1_ref, w2_ref, b2_ref,
              w3_ref, b3_ref, w4_ref, b4_ref, out_ref):
    e = emb_ref[...]
    s = s_ref[...]
    sum_d = jnp.dot(e, s, preferred_element_type=jnp.float32)
    sq_d = jnp.dot(e * e, s, preferred_element_type=jnp.float32)
    second = 0.5 * jnp.sum(sum_d * sum_d - sq_d, axis=1, keepdims=True)
    first = jnp.sum(fo_ref[...], axis=1, keepdims=True)
    h = jnp.maximum(jnp.dot(e, w1_ref[...], preferred_element_type=jnp.float32)
                    + b1_ref[...], 0.0)
    h = jnp.maximum(jnp.dot(h, w2_ref[...], preferred_element_type=jnp.float32)
                    + b2_ref[...], 0.0)
    h = jnp.maximum(jnp.dot(h, w3_ref[...], preferred_element_type=jnp.float32)
                    + b3_ref[...], 0.0)
    logit = (jnp.dot(h, w4_ref[...], preferred_element_type=jnp.float32)
             + b4_ref[...] + first + second)
    out_ref[...] = jax.nn.sigmoid(logit)


@jax.jit
def _tc_mlp(emb, fo, sel, W1, b1, W2, b2, W3, b3, W4, b4):
    full = lambda shape: pl.BlockSpec(shape, lambda i: (0, 0))
    return pl.pallas_call(
        _mlp_body,
        grid=(B // BB,),
        in_specs=[
            pl.BlockSpec((BB, F * D), lambda i: (i, 0)),
            pl.BlockSpec((BB, F), lambda i: (i, 0)),
            full(sel.shape),
            full(W1.shape), full(b1.shape),
            full(W2.shape), full(b2.shape),
            full(W3.shape), full(b3.shape),
            full(W4.shape), full(b4.shape),
        ],
        out_specs=pl.BlockSpec((BB, 1), lambda i: (i, 0)),
        out_shape=jax.ShapeDtypeStruct((B, 1), jnp.float32),
    )(emb, fo, sel, W1, b1, W2, b2, W3, b3, W4, b4)


_SEL = np.kron(np.ones((F, 1), np.float32), np.eye(D, dtype=np.float32))


def kernel(x, emb_tables, fo_tables, W1, b1, W2, b2, W3, b3, W4, b4):
    offs = jnp.arange(F, dtype=jnp.int32) * FSTRIDE
    gidx = (x.astype(jnp.int32) + offs[None, :]).reshape(NW * NGRP, GROWS)
    emb_nat = jnp.transpose(emb_tables, (0, 2, 1)).reshape(F * D, V + 1)
    emb_tail = jnp.pad(emb_nat[:, NFULL * VW:], ((0, 0), (0, VW - VTAIL)))
    emb_flat = _sc_transpose(emb_nat, emb_tail).reshape(ROWSP, D)
    fo_flat = jnp.pad(fo_tables.reshape(F, V + 1),
                      ((0, 0), (0, FSTRIDE - (V + 1)))).reshape(ROWSP)
    emb_g, fo_g = _sc_gather(gidx, emb_flat, fo_flat)
    out = _tc_mlp(emb_g.reshape(B, F * D), fo_g.reshape(B, F),
                  jnp.asarray(_SEL),
                  W1, b1.reshape(1, -1), W2, b2.reshape(1, -1),
                  W3, b3.reshape(1, -1), W4, b4.reshape(1, -1))
    return out.reshape(B)


# speedup vs baseline: 1.1215x; 1.1215x over previous
import functools

import jax
import jax.numpy as jnp
import numpy as np
from jax import lax
from jax.experimental import pallas as pl
from jax.experimental.pallas import tpu as pltpu
from jax.experimental.pallas import tpu_sc as plsc

B = 16384
F = 26
V = 100000
D = 16
ROWS = F * (V + 1)

NC = 2
NS = 16
NW = NC * NS
NIDX = B * F
CHUNK = 128
PER_W = NIDX // NW
NCH_W = PER_W // CHUNK
NCH_TOTAL = NIDX // CHUNK


GROUP = 13
GROWS = GROUP * CHUNK
NGRP = NCH_W // GROUP

VW = 1024
NFULL = (V + 1) // VW
VTAIL = (V + 1) - NFULL * VW
CPF = NFULL + 1
FSTRIDE = CPF * VW
ROWSP = F * FSTRIDE
NITEMS = F * CPF
K1_ITERS = -(-NITEMS // NW)


def _transpose_body(nat_hbm, tail_hbm, lin_out, ibuf, obuf, isem,
                    osem0, osem1):
    wid = lax.axis_index("s") * NC + lax.axis_index("c")
    row16 = lax.iota(jnp.int32, 16)
    osems = (osem0, osem1)

    def dst_off(item):
        return (item // CPF * FSTRIDE + item % CPF * VW) * D

    def issue_input(item, p):
        f = item // CPF
        c = item % CPF

        @pl.when(c < NFULL)
        def _():
            pltpu.async_copy(nat_hbm.at[pl.ds(f * D, D), pl.ds(c * VW, VW)],
                             ibuf.at[p], isem)

        @pl.when(c == NFULL)
        def _():
            pltpu.async_copy(tail_hbm.at[pl.ds(f * D, D)], ibuf.at[p], isem)

    def wait_input(item, p):
        f = item // CPF
        pltpu.make_async_copy(nat_hbm.at[pl.ds(f * D, D), pl.ds(0, VW)],
                              ibuf.at[p], isem).wait()

    def transpose_into(p):
        def xpose(v, cv):
            row = plsc.load_gather(ibuf.at[p], [row16, cv])
            obuf[p, pl.ds(v * 16, 16)] = row
            return cv + 1

        plsc.parallel_loop(0, VW, unroll=8,
                           carry=jnp.zeros((16,), jnp.int32))(xpose)

    def step(i, p):
        item = wid + i * NW
        nxt = item + NW
        prev = item - 2 * NW

        @pl.when(item < NITEMS)
        def _():
            wait_input(item, p)

        @pl.when(nxt < NITEMS)
        def _():
            issue_input(nxt, 1 - p)

        @pl.when((prev >= 0) & (prev < NITEMS))
        def _():
            pltpu.make_async_copy(obuf.at[p],
                                  lin_out.at[pl.ds(dst_off(prev), VW * D)],
                                  osems[p]).wait()

        @pl.when(item < NITEMS)
        def _():
            transpose_into(p)
            pltpu.async_copy(obuf.at[p],
                             lin_out.at[pl.ds(dst_off(item), VW * D)],
                             osems[p])

    issue_input(wid, 0)

    def outer(io, carry):
        for h in range(2):
            step(2 * io + h, h)
        return carry

    lax.fori_loop(0, (K1_ITERS + 2) // 2, outer, 0)


@jax.jit
def _sc_transpose(nat, tail):
    mesh = plsc.VectorSubcoreMesh(core_axis_name="c", subcore_axis_name="s")
    return pl.kernel(
        _transpose_body,
        out_type=jax.ShapeDtypeStruct((ROWSP * D,), jnp.float32),
        mesh=mesh,
        compiler_params=pltpu.CompilerParams(needs_layout_passes=False),
        scratch_types=[
            pltpu.VMEM((2, D, VW), jnp.float32),
            pltpu.VMEM((2, VW * D), jnp.float32),
            pltpu.SemaphoreType.DMA,
            pltpu.SemaphoreType.DMA,
            pltpu.SemaphoreType.DMA,
        ],
    )(nat, tail)


def _gather_body(gidx_hbm, emb_hbm, fo_hbm, emb_out, fo_out,
                 idx_v, ebig, fbig, gsem0, gsem1, wsem0, wsem1):
    wid = lax.axis_index("s") * NC + lax.axis_index("c")
    base_row = wid * PER_W
    pltpu.sync_copy(gidx_hbm.at[pl.ds(wid * NGRP, NGRP)], idx_v)

    gsems = (gsem0, gsem1)
    wsems = (wsem0, wsem1)

    def issue_gathers(g, p):
        pltpu.async_copy(emb_hbm.at[idx_v.at[g]], ebig.at[p], gsems[p])
        pltpu.async_copy(fo_hbm.at[idx_v.at[g]], fbig.at[p], gsems[p])

    def wait_gathers(g, p):
        pltpu.make_async_copy(emb_hbm.at[idx_v.at[g]], ebig.at[p],
                              gsems[p]).wait()
        pltpu.make_async_copy(fo_hbm.at[idx_v.at[g]], fbig.at[p],
                              gsems[p]).wait()

    def issue_wb(g, p):
        start = base_row + g * GROWS
        pltpu.async_copy(ebig.at[p], emb_out.at[pl.ds(start, GROWS)], wsems[p])
        pltpu.async_copy(fbig.at[p], fo_out.at[pl.ds(start, GROWS)], wsems[p])

    def wait_wb(g, p):
        start = base_row + g * GROWS
        pltpu.make_async_copy(ebig.at[p], emb_out.at[pl.ds(start, GROWS)],
                              wsems[p]).wait()
        pltpu.make_async_copy(fbig.at[p], fo_out.at[pl.ds(start, GROWS)],
                              wsems[p]).wait()

    for g in range(2):
        issue_gathers(g, g)
        wait_gathers(g, g)
        issue_wb(g, g)

    def outer(go, carry):
        for p in range(2):
            g = 2 * go + 2 + p
            wait_wb(g - 2, p)
            issue_gathers(g, p)
            wait_gathers(g, p)
            issue_wb(g, p)
        return carry

    lax.fori_loop(0, (NGRP - 2) // 2, outer, 0)

    for p in range(2):
        wait_wb(NGRP - 2 + p, p)


@jax.jit
def _sc_gather(gidx, emb_flat, fo_flat):
    mesh = plsc.VectorSubcoreMesh(core_axis_name="c", subcore_axis_name="s")
    return pl.kernel(
        _gather_body,
        out_type=(
            jax.ShapeDtypeStruct((NIDX, D), jnp.float32),
            jax.ShapeDtypeStruct((NIDX,), jnp.float32),
        ),
        mesh=mesh,
        compiler_params=pltpu.CompilerParams(use_tc_tiling_on_sc=False),
        scratch_types=[
            pltpu.VMEM((NGRP, GROWS), jnp.int32),
            pltpu.VMEM((2, GROWS, D), jnp.float32),
            pltpu.VMEM((2, GROWS), jnp.float32),
            pltpu.SemaphoreType.DMA,
            pltpu.SemaphoreType.DMA,
            pltpu.SemaphoreType.DMA,
            pltpu.SemaphoreType.DMA,
        ],
    )(gidx, emb_flat, fo_flat)


BB = 2048


def _mlp_body(emb_ref, fo_ref, s_ref, w1_ref, b1_ref, w2_ref, b2_ref,
              w3_ref, b3_ref, w4_ref, b4_ref, out_ref):
    e = emb_ref[...]
    s = s_ref[...]
    sum_d = jnp.dot(e, s, preferred_element_type=jnp.float32)
    sq_d = jnp.dot(e * e, s, preferred_element_type=jnp.float32)
    second = 0.5 * jnp.sum(sum_d * sum_d - sq_d, axis=1, keepdims=True)
    first = jnp.sum(fo_ref[...], axis=1, keepdims=True)
    h = jnp.maximum(jnp.dot(e, w1_ref[...], preferred_element_type=jnp.float32)
                    + b1_ref[...], 0.0)
    h = jnp.maximum(jnp.dot(h, w2_ref[...], preferred_element_type=jnp.float32)
                    + b2_ref[...], 0.0)
    h = jnp.maximum(jnp.dot(h, w3_ref[...], preferred_element_type=jnp.float32)
                    + b3_ref[...], 0.0)
    logit = (jnp.dot(h, w4_ref[...], preferred_element_type=jnp.float32)
             + b4_ref[...] + first + second)
    out_ref[...] = jax.nn.sigmoid(logit)


@jax.jit
def _tc_mlp(emb, fo, sel, W1, b1, W2, b2, W3, b3, W4, b4):
    full = lambda shape: pl.BlockSpec(shape, lambda i: (0, 0))
    return pl.pallas_call(
        _mlp_body,
        grid=(B // BB,),
        in_specs=[
            pl.BlockSpec((BB, F * D), lambda i: (i, 0)),
            pl.BlockSpec((BB, F), lambda i: (i, 0)),
            full(sel.shape),
            full(W1.shape), full(b1.shape),
            full(W2.shape), full(b2.shape),
            full(W3.shape), full(b3.shape),
            full(W4.shape), full(b4.shape),
        ],
        out_specs=pl.BlockSpec((BB, 1), lambda i: (i, 0)),
        out_shape=jax.ShapeDtypeStruct((B, 1), jnp.float32),
    )(emb, fo, sel, W1, b1, W2, b2, W3, b3, W4, b4)


_SEL = np.kron(np.ones((F, 1), np.float32), np.eye(D, dtype=np.float32))


def kernel(x, emb_tables, fo_tables, W1, b1, W2, b2, W3, b3, W4, b4):
    offs = jnp.arange(F, dtype=jnp.int32) * FSTRIDE
    gidx = (x.astype(jnp.int32) + offs[None, :]).reshape(NW * NGRP, GROWS)
    emb_nat = jnp.transpose(emb_tables, (0, 2, 1)).reshape(F * D, V + 1)
    emb_tail = jnp.pad(emb_nat[:, NFULL * VW:], ((0, 0), (0, VW - VTAIL)))
    emb_flat = _sc_transpose(emb_nat, emb_tail).reshape(ROWSP, D)
    fo_flat = jnp.pad(fo_tables.reshape(F, V + 1),
                      ((0, 0), (0, FSTRIDE - (V + 1)))).reshape(ROWSP)
    emb_g, fo_g = _sc_gather(gidx, emb_flat, fo_flat)
    out = _tc_mlp(emb_g.reshape(B, F * D), fo_g.reshape(B, F),
                  jnp.asarray(_SEL),
                  W1, b1.reshape(1, -1), W2, b2.reshape(1, -1),
                  W3, b3.reshape(1, -1), W4, b4.reshape(1, -1))
    return out.reshape(B)
